# baseline (speedup 1.0000x reference)
import functools

import jax
import jax.numpy as jnp
import numpy as np
from jax import lax
from jax.experimental import pallas as pl
from jax.experimental.pallas import tpu as pltpu
from jax.experimental.pallas import tpu_sc as plsc

H = W = 512
NUM = H * W
NPAIR = int(NUM * 0.15)
NTILE = 32

TP_CHUNK = 1024
TP_NCHUNK = NUM // NTILE // TP_CHUNK

CHUNK = 128
NP_T = -(-NPAIR // (NTILE * CHUNK)) * CHUNK
NCHUNK = NP_T // CHUNK

_COMPILER_PARAMS = pltpu.CompilerParams(
    needs_layout_passes=False, use_tc_tiling_on_sc=False)
_COMPILER_PARAMS_TILED = pltpu.CompilerParams(
    needs_layout_passes=False, use_tc_tiling_on_sc=True)
_MESH = plsc.VectorSubcoreMesh(core_axis_name="c", subcore_axis_name="s")


def _pair_partition():
    rng = np.random.RandomState(0)
    p1 = rng.choice(NUM, NPAIR, replace=True)
    rng.shuffle(p1)
    p2 = rng.choice(NUM, NPAIR, replace=True)
    rng.shuffle(p2)
    tot = NTILE * NP_T
    i1 = np.zeros(tot, np.int32)
    i2 = np.zeros(tot, np.int32)
    i1[:NPAIR] = p1
    i2[:NPAIR] = p2
    return (i1.reshape(NTILE, NCHUNK, CHUNK),
            i2.reshape(NTILE, NCHUNK, CHUNK))


_I1_NP, _I2_NP = _pair_partition()


def _tp_body(gt_hbm, pr_hbm, t_hbm, slab_a, slab_b, tch_a, tch_b,
             sem_in_a, sem_in_b, sem_out_a, sem_out_b):
    c = lax.axis_index("c")
    s = lax.axis_index("s")
    wid = s * 2 + c
    base_reg = wid * TP_NCHUNK
    iota = lax.iota(jnp.int32, 16)
    lane_hi = lax.shift_right_logical(iota, 3)
    lane_lo16 = (iota & 7) * 16

    def issue_slabs(ri, slab, sem):
        y0 = lax.shift_right_logical(ri, 2) * 8
        x0 = (ri & 3) * 128
        for b in range(8):
            pltpu.async_copy(
                gt_hbm.at[b, 0, pl.ds(y0, 8), pl.ds(x0, 128)],
                slab.at[b], sem)
            pltpu.async_copy(
                pr_hbm.at[b, 0, pl.ds(y0, 8), pl.ds(x0, 128)],
                slab.at[b + 8], sem)

    def wait_slabs(slab, sem):
        for k in range(16):
            pltpu.make_async_copy(
                gt_hbm.at[0, 0, pl.ds(0, 8), pl.ds(0, 128)],
                slab.at[k], sem).wait()

    def compute(ri, slab, tch, sem_out):
        def group(g, _):
            dy = lax.shift_right_logical(g, 3)
            xg = g & 7
            dyv = jnp.full((16,), 0, jnp.int32) + dy
            trow = xg * 2 + lane_hi
            for k in range(16):
                v = slab[k, dy, pl.ds(xg * 16, 16)]
                plsc.store_scatter(tch, [dyv, trow, lane_lo16 + k], v)
            return 0
        lax.fori_loop(0, 64, group, 0)
        y0 = lax.shift_right_logical(ri, 2) * 8
        x0r = (ri & 3) * 16
        for dy in range(8):
            tr0 = (y0 + dy) * 64 + x0r
            pltpu.async_copy(tch.at[dy], t_hbm.at[pl.ds(tr0, 16), :],
                             sem_out)

    def wait_out(tch, sem):
        for dy in range(8):
            pltpu.make_async_copy(t_hbm.at[pl.ds(0, 16), :], tch.at[dy],
                                  sem).wait()

    issue_slabs(base_reg, slab_a, sem_in_a)

    def loop(i, carry):
        r0 = base_reg + 2 * i
        issue_slabs(r0 + 1, slab_b, sem_in_b)
        wait_slabs(slab_a, sem_in_a)

        @pl.when(i >= 1)
        def _():
            wait_out(tch_a, sem_out_a)

        compute(r0, slab_a, tch_a, sem_out_a)

        @pl.when(i < (TP_NCHUNK // 2) - 1)
        def _():
            issue_slabs(r0 + 2, slab_a, sem_in_a)

        wait_slabs(slab_b, sem_in_b)

        @pl.when(i >= 1)
        def _():
            wait_out(tch_b, sem_out_b)

        compute(r0 + 1, slab_b, tch_b, sem_out_b)
        return carry

    lax.fori_loop(0, TP_NCHUNK // 2, loop, 0)
    wait_out(tch_a, sem_out_a)
    wait_out(tch_b, sem_out_b)


_sc_transpose = functools.partial(
    pl.kernel,
    mesh=_MESH,
    compiler_params=_COMPILER_PARAMS_TILED,
    out_type=jax.ShapeDtypeStruct((NUM // 8, 128), jnp.float32),
    scratch_types=[
        pltpu.VMEM((16, 8, 128), jnp.float32),
        pltpu.VMEM((16, 8, 128), jnp.float32),
        pltpu.VMEM((8, 16, 128), jnp.float32),
        pltpu.VMEM((8, 16, 128), jnp.float32),
        pltpu.SemaphoreType.DMA,
        pltpu.SemaphoreType.DMA,
        pltpu.SemaphoreType.DMA,
        pltpu.SemaphoreType.DMA,
    ],
)(_tp_body)


def _pair_compute(b1f, b2f, acc, iota):
    for g in range(CHUNK // 16):
        rowi = g * 16 + iota
        ls = []
        for b in range(8):
            cb = jnp.full((16,), b, jnp.int32)
            cq = jnp.full((16,), b + 8, jnp.int32)
            g1 = plsc.load_gather(b1f, [rowi, cb])
            g2 = plsc.load_gather(b2f, [rowi, cb])
            q1 = plsc.load_gather(b1f, [rowi, cq])
            q2 = plsc.load_gather(b2f, [rowi, cq])
            gd = g1 - g2
            pd = q1 - q2
            ls.append(jnp.where(gd - gd == 0.0, jnp.abs(gd - pd), 0.0))
        tot = ls[0]
        for b in range(1, 8):
            tot = tot + ls[b]
        lo = [jnp.minimum(ls[2 * i], ls[2 * i + 1]) for i in range(4)]
        hi = [jnp.maximum(ls[2 * i], ls[2 * i + 1]) for i in range(4)]
        m1l = jnp.minimum(lo[0], lo[1])
        m1h = jnp.minimum(jnp.maximum(lo[0], lo[1]),
                          jnp.minimum(hi[0], hi[1]))
        m2l = jnp.minimum(lo[2], lo[3])
        m2h = jnp.minimum(jnp.maximum(lo[2], lo[3]),
                          jnp.minimum(hi[2], hi[3]))
        f1 = jnp.minimum(m1l, m2l)
        f2 = jnp.minimum(jnp.maximum(m1l, m2l), jnp.minimum(m1h, m2h))
        acc = acc + (tot - f1 - f2)
    return acc


def _sc_body(t_hbm, i1_hbm, i2_hbm, out_hbm,
             i1_v, i2_v, b1_a, b1_b, b2_a, b2_b, row_v, slab16, shared,
             ssa, ssb, spa, spb):
    c = lax.axis_index("c")
    s = lax.axis_index("s")
    wid = s * 2 + c

    pltpu.sync_copy(i1_hbm.at[wid], i1_v)
    pltpu.sync_copy(i2_hbm.at[wid], i2_v)

    iota = lax.iota(jnp.int32, 16)

    def issue(ci, b1, b2, ssem, psem):
        pltpu.async_copy(t_hbm.at[i1_v.at[ci]], b1, ssem)
        pltpu.async_copy(t_hbm.at[i2_v.at[ci]], b2, psem)

    def wait(b1, b2, ssem, psem):
        pltpu.make_async_copy(t_hbm.at[i1_v.at[0]], b1, ssem).wait()
        pltpu.make_async_copy(t_hbm.at[i2_v.at[0]], b2, psem).wait()

    issue(0, b1_a, b2_a, ssa, spa)

    def loop(i, acc):
        c0 = 2 * i
        issue(c0 + 1, b1_b, b2_b, ssb, spb)
        wait(b1_a, b2_a, ssa, spa)
        acc = _pair_compute(b1_a, b2_a, acc, iota)

        @pl.when(i < (NCHUNK // 2) - 1)
        def _():
            issue(c0 + 2, b1_a, b2_a, ssa, spa)

        wait(b1_b, b2_b, ssb, spb)
        acc = _pair_compute(b1_b, b2_b, acc, iota)
        return acc

    acc = lax.fori_loop(0, NCHUNK // 2, loop, jnp.zeros((16,), jnp.float32))

    row_v[0, :] = acc
    pltpu.sync_copy(row_v, shared.at[pl.ds(s, 1), :])
    plsc.subcore_barrier()

    @pl.when(s == 0)
    def _():
        pltpu.sync_copy(shared, slab16)
        tot = slab16[0, :]
        for r in range(1, 16):
            tot = tot + slab16[r, :]
        row_v[0, :] = tot
        pltpu.sync_copy(row_v, out_hbm.at[c])


_sc_pairloss = functools.partial(
    pl.kernel,
    mesh=_MESH,
    compiler_params=_COMPILER_PARAMS,
    out_type=jax.ShapeDtypeStruct((2, 1, 16), jnp.float32),
    scratch_types=[
        pltpu.VMEM((NCHUNK, CHUNK), jnp.int32),
        pltpu.VMEM((NCHUNK, CHUNK), jnp.int32),
        pltpu.VMEM((CHUNK, 16), jnp.float32),
        pltpu.VMEM((CHUNK, 16), jnp.float32),
        pltpu.VMEM((CHUNK, 16), jnp.float32),
        pltpu.VMEM((CHUNK, 16), jnp.float32),
        pltpu.VMEM((1, 16), jnp.float32),
        pltpu.VMEM((16, 16), jnp.float32),
        pltpu.VMEM_SHARED((16, 16), jnp.float32),
        pltpu.SemaphoreType.DMA,
        pltpu.SemaphoreType.DMA,
        pltpu.SemaphoreType.DMA,
        pltpu.SemaphoreType.DMA,
    ],
)(_sc_body)


def kernel(gt_depth, pred_depth):
    table = _sc_transpose(gt_depth, pred_depth)
    i1 = jnp.asarray(_I1_NP)
    i2 = jnp.asarray(_I2_NP)
    parts = _sc_pairloss(table.reshape(NUM, 16), i1, i2)
    return jnp.sum(parts) * np.float32(1.0 / (6 * NPAIR))
